# Initial kernel scaffold; baseline (speedup 1.0000x reference)
import jax
import jax.numpy as jnp
from jax.experimental import pallas as pl


def kernel(x, edge_index, edge_weight, W1, b1, W2, b2):
    raise NotImplementedError("write your pallas kernel here")


# speedup vs baseline: 16.1482x; 16.1482x over previous
import functools

import jax
import jax.numpy as jnp
from jax import lax
from jax.experimental import pallas as pl
from jax.experimental.pallas import tpu as pltpu
from jax.experimental.pallas import tpu_sc as plsc

_NC = 2
_NS = 16
_NW = _NC * _NS
_L = 16

_CHUNK = 128
_QS = 8
_SUPER = _CHUNK * _QS


def _round_up(v, m):
    return (v + m - 1) // m * m


def _spmm_sc(h, src2d, dst2d, w2d, npad):
    nchunks = src2d.shape[0]
    per_w_chunks = nchunks // _NW
    nsuper = per_w_chunks // _QS
    rpt = npad // _NS

    mesh = plsc.VectorSubcoreMesh(core_axis_name="c", subcore_axis_name="s")

    @functools.partial(
        pl.kernel,
        out_type=jax.ShapeDtypeStruct((_NC, npad, _L), jnp.float32),
        mesh=mesh,
        scratch_types=[
            pltpu.VMEM((2, _QS, _CHUNK), jnp.int32),
            pltpu.VMEM((2, _QS, _CHUNK), jnp.int32),
            pltpu.VMEM((2, _QS, _CHUNK), jnp.float32),
            pltpu.VMEM((2, _QS, _CHUNK, _L), jnp.float32),
            pltpu.VMEM((rpt, _L), jnp.float32),
            pltpu.VMEM_SHARED((npad, _L), jnp.float32),
            pltpu.SemaphoreType.DMA,
            pltpu.SemaphoreType.DMA,
            pltpu.SemaphoreType.DMA,
            pltpu.SemaphoreType.DMA,
        ],
        compiler_params=pltpu.CompilerParams(use_tc_tiling_on_sc=False),
    )
    def spmm(h_hbm, src_hbm, dst_hbm, w_hbm, out_hbm,
             srcb, dstb, wb, rowsb, stage, acc, g0, g1, s0, s1):
        c = lax.axis_index("c")
        s = lax.axis_index("s")
        wid = c * _NS + s
        chunk0 = wid * per_w_chunks
        gsem = (g0, g1)
        ssem = (s0, s1)

        zero = jnp.zeros((_L,), jnp.float32)

        @pl.loop(0, rpt)
        def _(i):
            stage[i, :] = zero

        pltpu.sync_copy(stage, acc.at[pl.ds(s * rpt, rpt)])
        plsc.subcore_barrier()

        gds = [None, None]
        sds = [None, None]

        def stage_in(slot, t):
            off = chunk0 + t * _QS
            pltpu.sync_copy(src_hbm.at[pl.ds(off, _QS)], srcb.at[slot])
            pltpu.sync_copy(dst_hbm.at[pl.ds(off, _QS)], dstb.at[slot])
            pltpu.sync_copy(w_hbm.at[pl.ds(off, _QS)], wb.at[slot])
            gds[slot] = [
                pltpu.async_copy(h_hbm.at[srcb.at[slot, q]],
                                 rowsb.at[slot, q], gsem[slot])
                for q in range(_QS)
            ]

        def scale(slot):
            for q in range(_QS):

                @pl.loop(0, _CHUNK // _L)
                def _(g):
                    wvec = wb[slot, q, pl.ds(g * _L, _L)]
                    for i in range(_L):
                        wv = wvec[i]
                        eidx = g * _L + i
                        rowsb[slot, q, eidx, :] = rowsb[slot, q, eidx, :] * wv

        def fire_scatter(slot):
            sds[slot] = [
                pltpu.async_copy(rowsb.at[slot, q],
                                 acc.at[dstb.at[slot, q]],
                                 ssem[slot], add=True)
                for q in range(_QS)
            ]

        stage_in(0, 0)
        for t in range(nsuper):
            slot = t & 1
            if t + 1 < nsuper:
                if sds[slot ^ 1] is not None:
                    for d in sds[slot ^ 1]:
                        d.wait()
                    sds[slot ^ 1] = None
                stage_in(slot ^ 1, t + 1)
            for d in gds[slot]:
                d.wait()
            gds[slot] = None
            scale(slot)
            fire_scatter(slot)
        for slot in (0, 1):
            if sds[slot] is not None:
                for d in sds[slot]:
                    d.wait()
                sds[slot] = None

        plsc.subcore_barrier()
        pltpu.sync_copy(acc.at[pl.ds(s * rpt, rpt)], stage)
        pltpu.sync_copy(stage, out_hbm.at[c, pl.ds(s * rpt, rpt)])

    return spmm(h, src2d, dst2d, w2d)


def _mm_body(a_ref, b_ref, o_ref):
    o_ref[...] = jnp.dot(a_ref[...], b_ref[...],
                         preferred_element_type=jnp.float32)


def _dense_mm(a, b):
    return pl.pallas_call(
        _mm_body,
        out_shape=jax.ShapeDtypeStruct((a.shape[0], b.shape[1]), jnp.float32),
    )(a, b)


def _layer2_body(p_ref, b1_ref, w2_ref, o_ref):
    hid = jnp.maximum(p_ref[0] + p_ref[1] + b1_ref[...], 0.0)
    o_ref[...] = jnp.dot(hid, w2_ref[...], preferred_element_type=jnp.float32)


def _layer2(partials, b1row, w2pad):
    npad = partials.shape[1]
    return pl.pallas_call(
        _layer2_body,
        out_shape=jax.ShapeDtypeStruct((npad, _L), jnp.float32),
    )(partials, b1row, w2pad)


def _final_body(p_ref, b2_ref, o_ref):
    o_ref[...] = p_ref[0] + p_ref[1] + b2_ref[...]


def _final(partials, b2row):
    npad = partials.shape[1]
    return pl.pallas_call(
        _final_body,
        out_shape=jax.ShapeDtypeStruct((npad, _L), jnp.float32),
    )(partials, b2row)


def kernel(x, edge_index, edge_weight, W1, b1, W2, b2):
    n, d = x.shape
    h1w = W1.shape[1]
    h2w = W2.shape[1]
    e = edge_index.shape[1]

    npad = _round_up(n, _NS * 8)

    per_w = _round_up(_round_up(e, _NW) // _NW, _SUPER)
    epad = _NW * per_w
    pad = epad - e
    src2d = jnp.pad(edge_index[0].astype(jnp.int32), (0, pad)).reshape(
        -1, _CHUNK)
    dst2d = jnp.pad(edge_index[1].astype(jnp.int32), (0, pad)).reshape(
        -1, _CHUNK)
    w2d = jnp.pad(edge_weight.astype(jnp.float32), (0, pad)).reshape(
        -1, _CHUNK)

    w1pad = jnp.pad(W1, ((0, 0), (0, _L - h1w)))
    b1row = jnp.pad(b1, (0, _L - h1w)).reshape(1, _L)
    w2pad = jnp.pad(W2, ((0, _L - h1w), (0, _L - h2w)))
    b2row = jnp.pad(b2, (0, _L - h2w)).reshape(1, _L)

    s1 = _dense_mm(x, w1pad)
    p1 = _spmm_sc(s1, src2d, dst2d, w2d, npad)
    s2 = _layer2(p1, b1row, w2pad)
    p2 = _spmm_sc(s2, src2d, dst2d, w2d, npad)
    outp = _final(p2, b2row)
    return outp[:n, :h2w]
